# baseline (speedup 1.0000x reference)
import functools

import jax
import jax.numpy as jnp
from jax import lax
from jax.experimental import pallas as pl
from jax.experimental.pallas import tpu as pltpu
from jax.experimental.pallas import tpu_sc as plsc

NC = 2
NS = 16
NW = NC * NS
L = 16
IDX_CHUNK = 128


@functools.lru_cache(maxsize=None)
def _build(B, K):
    assert B % (8 * NW) == 0
    b_per_w = B // NW
    n_chunks = b_per_w // IDX_CHUNK
    n_k = K // L
    mesh = plsc.VectorSubcoreMesh(core_axis_name="c", subcore_axis_name="s")

    @functools.partial(
        pl.kernel,
        out_type=jax.ShapeDtypeStruct((B,), jnp.float32),
        mesh=mesh,
        compiler_params=pltpu.CompilerParams(
            needs_layout_passes=False, use_tc_tiling_on_sc=False),
        scratch_types=[
            pltpu.VMEM((n_chunks, IDX_CHUNK), jnp.int32),
            pltpu.VMEM((n_chunks, IDX_CHUNK), jnp.int32),
            pltpu.VMEM((n_chunks, IDX_CHUNK), jnp.int32),
            pltpu.VMEM((b_per_w, K), jnp.float32),
            pltpu.VMEM((b_per_w, K), jnp.float32),
            pltpu.VMEM((b_per_w, K), jnp.float32),
            pltpu.VMEM((b_per_w,), jnp.float32),
            pltpu.VMEM((L, L), jnp.float32),
            pltpu.SemaphoreType.DMA,
        ],
    )
    def parafac(idx0_h, idx1_h, idx2_h, f0, f1, f2, out,
                idx0, idx1, idx2, rows0, rows1, rows2, out_v, acc16, sem):
        wid = lax.axis_index("s") * NC + lax.axis_index("c")
        row0 = wid * n_chunks
        pltpu.sync_copy(idx0_h.at[pl.ds(row0, n_chunks)], idx0)
        pltpu.sync_copy(idx1_h.at[pl.ds(row0, n_chunks)], idx1)
        pltpu.sync_copy(idx2_h.at[pl.ds(row0, n_chunks)], idx2)
        copies = []
        for c in range(n_chunks):
            sl = pl.ds(c * IDX_CHUNK, IDX_CHUNK)
            copies.append(pltpu.async_copy(f0.at[idx0.at[c]], rows0.at[sl], sem))
            copies.append(pltpu.async_copy(f1.at[idx1.at[c]], rows1.at[sl], sem))
            copies.append(pltpu.async_copy(f2.at[idx2.at[c]], rows2.at[sl], sem))
        for cp in copies:
            cp.wait()

        lane_iota = lax.iota(jnp.int32, L)

        def group(g, carry):
            for lb in range(L):
                b = g * L + lb
                acc = (rows0[b, pl.ds(0, L)] * rows1[b, pl.ds(0, L)]
                       * rows2[b, pl.ds(0, L)])
                for j in range(1, n_k):
                    sl = pl.ds(j * L, L)
                    acc = acc + rows0[b, sl] * rows1[b, sl] * rows2[b, sl]
                acc16[lb, :] = acc
            tot = plsc.load_gather(acc16, [lane_iota, jnp.zeros((L,), jnp.int32)])
            for col in range(1, L):
                tot = tot + plsc.load_gather(
                    acc16, [lane_iota, jnp.full((L,), col, jnp.int32)])
            out_v[pl.ds(g * L, L)] = tot
            return carry

        lax.fori_loop(0, b_per_w // L, group, 0)
        pltpu.sync_copy(out_v, out.at[pl.ds(wid * b_per_w, b_per_w)])

    return parafac


def kernel(indices, F0, F1, F2):
    B = indices.shape[0]
    K = F0.shape[1]
    idx = indices.astype(jnp.int32)
    i0 = idx[:, 0].reshape(B // IDX_CHUNK, IDX_CHUNK)
    i1 = idx[:, 1].reshape(B // IDX_CHUNK, IDX_CHUNK)
    i2 = idx[:, 2].reshape(B // IDX_CHUNK, IDX_CHUNK)
    return _build(B, K)(i0, i1, i2, F0, F1, F2)
